# Initial kernel scaffold; baseline (speedup 1.0000x reference)
import jax
import jax.numpy as jnp
from jax.experimental import pallas as pl


def kernel(x, edge_index, W1, b1, W2, b2, Wc, bc):
    raise NotImplementedError("write your pallas kernel here")


# speedup vs baseline: 10.5011x; 10.5011x over previous
import functools

import jax
import jax.numpy as jnp
from jax import lax
from jax.experimental import pallas as pl
from jax.experimental.pallas import tpu as pltpu
from jax.experimental.pallas import tpu_sc as plsc

_NC = 2
_NS = 16
_NW = _NC * _NS
_K = 80
_L = 16
_SH = 14


def _sc_mesh():
    return plsc.VectorSubcoreMesh(core_axis_name="c", subcore_axis_name="s")


def _sc_params():
    return pltpu.CompilerParams(needs_layout_passes=False)


def _row_split(n):
    ch = (n // _NS) // 8 * 8
    tail = n - _NS * ch
    assert ch % 8 == 0 and tail % 8 == 0 and tail >= 0
    return ch, tail


def _make_part(E, N):
    epw = E // _NW
    nj = epw // _K
    half = N // 2
    cap = epw + _L
    assert epw * _NW == E and nj * _K == epw and cap % 8 == 0
    assert N < (1 << _SH) and half % 8 == 0

    out_types = (
        jax.ShapeDtypeStruct((2 * _NW * cap,), jnp.int32),
        jax.ShapeDtypeStruct((_NW * _L,), jnp.int32),
    )

    @functools.partial(
        pl.kernel,
        out_type=out_types,
        mesh=_sc_mesh(),
        compiler_params=_sc_params(),
        scratch_types=[
            pltpu.VMEM((nj, _K), jnp.int32),
            pltpu.VMEM((nj, _K), jnp.int32),
            pltpu.VMEM((cap,), jnp.int32),
            pltpu.VMEM((cap,), jnp.int32),
            pltpu.VMEM((_L,), jnp.int32),
        ],
    )
    def part_k(src_hbm, dst_hbm, pk_hbm, cnt_hbm, sbuf, dbuf, pa, pb, cnt_v):
        c = lax.axis_index("c")
        s = lax.axis_index("s")
        wid = s * _NC + c
        pltpu.sync_copy(src_hbm.at[wid], sbuf)
        pltpu.sync_copy(dst_hbm.at[wid], dbuf)

        pad_pk = lax.iota(jnp.int32, _L) * 577 + (half << _SH)

        def pre(i, carry):
            o = pl.multiple_of(i * _L, 8)
            pa[pl.ds(o, _L)] = pad_pk
            pb[pl.ds(o, _L)] = pad_pk
            return carry

        lax.fori_loop(0, cap // _L, pre, 0)

        def body(j, carry):
            offa, offb = carry
            for k in range(_K // _L):
                sv = sbuf[j, pl.ds(k * _L, _L)]
                dv = dbuf[j, pl.ds(k * _L, _L)]
                ma = dv < half
                mb = jnp.logical_not(ma)
                mai = ma.astype(jnp.int32)
                mbi = mb.astype(jnp.int32)
                ia = offa + plsc.cumsum(mai) - mai
                ib = offb + plsc.cumsum(mbi) - mbi
                dl = jnp.where(ma, dv, dv - half)
                pk = sv | (dl << _SH)
                plsc.store_scatter(pa, [ia], pk, mask=ma)
                plsc.store_scatter(pb, [ib], pk, mask=mb)
                na = jnp.sum(mai)
                offa = offa + na
                offb = offb + (_L - na)
            return offa, offb

        offa, offb = lax.fori_loop(0, nj, body, (0, 0))

        pltpu.sync_copy(pa, pk_hbm.at[pl.ds(wid * cap, cap)])
        pltpu.sync_copy(pb, pk_hbm.at[pl.ds((_NW + wid) * cap, cap)])
        io = lax.iota(jnp.int32, _L)
        cnt_v[...] = jnp.where(io == 0, offa, jnp.where(io == 1, offb, 0))
        pltpu.sync_copy(cnt_v, cnt_hbm.at[pl.ds(wid * _L, _L)])

    return part_k


def _make_scatter(E, N, D):
    epw = E // _NW
    half = N // 2
    cap = epw + _L
    nacc = half + 8
    ach, atail = _row_split(half)
    mask14 = (1 << _SH) - 1

    @functools.partial(
        pl.kernel,
        out_type=jax.ShapeDtypeStruct((N, D), jnp.float32),
        mesh=_sc_mesh(),
        compiler_params=_sc_params(),
        scratch_types=[
            pltpu.VMEM((_K,), jnp.int32),
            pltpu.VMEM((_K,), jnp.int32),
            pltpu.VMEM((_K,), jnp.int32),
            pltpu.VMEM((_K, D), jnp.float32),
            pltpu.VMEM((_L,), jnp.int32),
            pltpu.VMEM_SHARED((nacc, D), jnp.float32),
            pltpu.SemaphoreType.DMA,
        ],
    )
    def scat_k(y_hbm, pk_hbm, cnt_hbm, zeros_hbm, out_hbm, pk_v, si_v, di_v,
               rows_v, cnt_v, acc_sh, sem):
        c = lax.axis_index("c")
        s = lax.axis_index("s")
        row0 = pl.multiple_of(s * ach, 8)
        pltpu.sync_copy(zeros_hbm, acc_sh.at[pl.ds(row0, ach)])

        @pl.when(s == _NS - 1)
        def _zt():
            pltpu.sync_copy(zeros_hbm.at[pl.ds(0, atail)],
                            acc_sh.at[pl.ds(half - atail, atail)])

        plsc.subcore_barrier()

        for r in range(2):
            w = 2 * s + r
            pltpu.sync_copy(cnt_hbm.at[pl.ds(w * _L, _L)], cnt_v)
            cvec = cnt_v[...]
            cnt = jnp.where(c == 0, cvec[0], cvec[1])
            nch = lax.div(cnt + (_K - 1), _K)
            regbase = (c * _NW + w) * cap

            def body(i, carry):
                base = pl.multiple_of(regbase + i * _K, 8)
                pltpu.sync_copy(pk_hbm.at[pl.ds(base, _K)], pk_v)
                for k in range(_K // _L):
                    pk = pk_v[pl.ds(k * _L, _L)]
                    si_v[pl.ds(k * _L, _L)] = pk & mask14
                    di_v[pl.ds(k * _L, _L)] = lax.shift_right_logical(pk, _SH)
                pltpu.async_copy(y_hbm.at[si_v], rows_v, sem).wait()
                pltpu.sync_copy(rows_v, acc_sh.at[di_v], add=True)
                return carry

            lax.fori_loop(0, nch, body, 0)

        plsc.subcore_barrier()
        out0 = pl.multiple_of(c * half + s * ach, 8)
        pltpu.sync_copy(acc_sh.at[pl.ds(row0, ach)],
                        out_hbm.at[pl.ds(out0, ach)])

        @pl.when(s == _NS - 1)
        def _ct():
            pltpu.sync_copy(acc_sh.at[pl.ds(half - atail, atail)],
                            out_hbm.at[pl.ds(c * half + half - atail, atail)])

    return scat_k


_R = 1000


def _dinv_block(degp_ref):
    return lax.rsqrt(degp_ref[...][:, 0:1] + 1.0)


def _tc_linear_body(h_ref, w_ref, degp_ref, fdeg_ref, y_ref):
    dinv = _dinv_block(degp_ref)
    y = jnp.dot(h_ref[...], w_ref[...],
                preferred_element_type=jnp.float32,
                precision=lax.Precision.HIGHEST) * dinv
    y_ref[...] = jnp.where(fdeg_ref[...] > 0.0, 1.0, y)


def _tc_post_body(acc_ref, y_ref, degp_ref, h_ref, b_ref, frelu_ref,
                  fdeg_ref, hn_ref, degn_ref):
    dinv = _dinv_block(degp_ref)
    hp = dinv * (acc_ref[...] + y_ref[...]) + b_ref[...]
    hp = jnp.where(frelu_ref[...] > 0.0, jnp.maximum(hp, 0.0), hp)
    isdeg = fdeg_ref[...] > 0.0
    hn_ref[...] = jnp.where(isdeg, h_ref[...], hp)
    degn_ref[...] = jnp.where(isdeg, acc_ref[...], degp_ref[...])


def _tc_head_body(h_ref, wc_ref, bc_ref, out_ref):
    out_ref[...] = jnp.dot(h_ref[...], wc_ref[...],
                           preferred_element_type=jnp.float32,
                           precision=lax.Precision.HIGHEST) + bc_ref[...]


def _row_spec(D):
    return pl.BlockSpec((_R, D), lambda i: (i, 0))


def _full_spec(shape):
    nd = len(shape)
    return pl.BlockSpec(shape, lambda i, _nd=nd: (0,) * _nd)


def _tc_linear(h, W, degp, fdeg):
    N, D = h.shape
    return pl.pallas_call(
        _tc_linear_body,
        grid=(N // _R,),
        in_specs=[_row_spec(D), _full_spec((D, D)), _row_spec(D),
                  _full_spec((1, D))],
        out_specs=_row_spec(D),
        out_shape=jax.ShapeDtypeStruct((N, D), jnp.float32),
    )(h, W, degp, fdeg)


def _tc_post(acc, y, degp, h, b, frelu, fdeg):
    N, D = y.shape
    return pl.pallas_call(
        _tc_post_body,
        grid=(N // _R,),
        in_specs=[_row_spec(D), _row_spec(D), _row_spec(D), _row_spec(D),
                  _full_spec((1, D)), _full_spec((1, D)), _full_spec((1, D))],
        out_specs=[_row_spec(D), _row_spec(D)],
        out_shape=[jax.ShapeDtypeStruct((N, D), jnp.float32),
                   jax.ShapeDtypeStruct((N, D), jnp.float32)],
    )(acc, y, degp, h, b, frelu, fdeg)


def _tc_head(h, Wc, bc):
    N, D = h.shape
    return pl.pallas_call(
        _tc_head_body,
        grid=(N // _R,),
        in_specs=[_row_spec(D), _full_spec((D, D)), _full_spec((1, D))],
        out_specs=_row_spec(D),
        out_shape=jax.ShapeDtypeStruct((N, D), jnp.float32),
    )(h, Wc, bc)


def kernel(x, edge_index, W1, b1, W2, b2, Wc, bc):
    N, D = x.shape
    E = edge_index.shape[1]
    epw = E // _NW
    src3 = edge_index[0].reshape(_NW, epw // _K, _K)
    dst3 = edge_index[1].reshape(_NW, epw // _K, _K)
    ach, _ = _row_split(N // 2)

    part_fn = _make_part(E, N)
    scat_fn = _make_scatter(E, N, D)

    zacc = jnp.zeros((ach, D), jnp.float32)
    pk, cnts = part_fn(src3, dst3)

    one_row = jnp.ones((1, D), jnp.float32)
    zero_row = jnp.zeros((1, D), jnp.float32)
    w_st = jnp.stack([W1, W1, W2])
    b_st = jnp.stack([zero_row, b1.reshape(1, D), b2.reshape(1, D)])
    frelu_st = jnp.stack([zero_row, one_row, zero_row])
    fdeg_st = jnp.stack([one_row, zero_row, zero_row])

    def step(carry, xs):
        h, degp = carry
        W, b, frelu, fdeg = xs
        y = _tc_linear(h, W, degp, fdeg)
        acc = scat_fn(y, pk, cnts, zacc)
        hn, degn = _tc_post(acc, y, degp, h, b, frelu, fdeg)
        return (hn, degn), None

    degp0 = jnp.zeros((N, D), jnp.float32)
    (h, _), _ = lax.scan(step, (x, degp0), (w_st, b_st, frelu_st, fdeg_st))
    out = _tc_head(h, Wc, bc.reshape(1, D))
    return (h, out)


# speedup vs baseline: 21.0956x; 2.0089x over previous
import functools

import jax
import jax.numpy as jnp
from jax import lax
from jax.experimental import pallas as pl
from jax.experimental.pallas import tpu as pltpu
from jax.experimental.pallas import tpu_sc as plsc

_NC = 2
_NS = 16
_NW = _NC * _NS
_K = 80
_KC = 128
_L = 16
_SH = 14


def _sc_mesh():
    return plsc.VectorSubcoreMesh(core_axis_name="c", subcore_axis_name="s")


def _sc_params():
    return pltpu.CompilerParams(needs_layout_passes=False)


def _row_split(n):
    ch = (n // _NS) // 8 * 8
    tail = n - _NS * ch
    assert ch % 8 == 0 and tail % 8 == 0 and tail >= 0
    return ch, tail


def _make_part(E, N):
    epw = E // _NW
    nj = epw // _K
    half = N // 2
    cap = epw + _KC
    assert epw * _NW == E and nj * _K == epw and cap % 8 == 0
    assert N < (1 << _SH) and half % 8 == 0

    out_types = (
        jax.ShapeDtypeStruct((2 * _NW * cap,), jnp.int32),
        jax.ShapeDtypeStruct((_NW * _L,), jnp.int32),
    )

    @functools.partial(
        pl.kernel,
        out_type=out_types,
        mesh=_sc_mesh(),
        compiler_params=_sc_params(),
        scratch_types=[
            pltpu.VMEM((nj, _K), jnp.int32),
            pltpu.VMEM((nj, _K), jnp.int32),
            pltpu.VMEM((cap,), jnp.int32),
            pltpu.VMEM((cap,), jnp.int32),
            pltpu.VMEM((_L,), jnp.int32),
        ],
    )
    def part_k(src_hbm, dst_hbm, pk_hbm, cnt_hbm, sbuf, dbuf, pa, pb, cnt_v):
        c = lax.axis_index("c")
        s = lax.axis_index("s")
        wid = s * _NC + c
        pltpu.sync_copy(src_hbm.at[wid], sbuf)
        pltpu.sync_copy(dst_hbm.at[wid], dbuf)

        pad_pk = lax.iota(jnp.int32, _L) * 577 + (half << _SH)

        def pre(i, carry):
            o = pl.multiple_of(i * _L, 8)
            pa[pl.ds(o, _L)] = pad_pk
            pb[pl.ds(o, _L)] = pad_pk
            return carry

        lax.fori_loop(0, cap // _L, pre, 0)

        def body(j, carry):
            offa, offb = carry
            for k in range(_K // _L):
                sv = sbuf[j, pl.ds(k * _L, _L)]
                dv = dbuf[j, pl.ds(k * _L, _L)]
                ma = dv < half
                mb = jnp.logical_not(ma)
                mai = ma.astype(jnp.int32)
                mbi = mb.astype(jnp.int32)
                ia = offa + plsc.cumsum(mai) - mai
                ib = offb + plsc.cumsum(mbi) - mbi
                dl = jnp.where(ma, dv, dv - half)
                pk = sv | (dl << _SH)
                plsc.store_scatter(pa, [ia], pk, mask=ma)
                plsc.store_scatter(pb, [ib], pk, mask=mb)
                na = jnp.sum(mai)
                offa = offa + na
                offb = offb + (_L - na)
            return offa, offb

        offa, offb = lax.fori_loop(0, nj, body, (0, 0))

        pltpu.sync_copy(pa, pk_hbm.at[pl.ds(wid * cap, cap)])
        pltpu.sync_copy(pb, pk_hbm.at[pl.ds((_NW + wid) * cap, cap)])
        io = lax.iota(jnp.int32, _L)
        cnt_v[...] = jnp.where(io == 0, offa, jnp.where(io == 1, offb, 0))
        pltpu.sync_copy(cnt_v, cnt_hbm.at[pl.ds(wid * _L, _L)])

    return part_k


def _make_scatter(E, N, D):
    epw = E // _NW
    half = N // 2
    cap = epw + _KC
    nacc = half + 8
    ach, atail = _row_split(half)
    mask14 = (1 << _SH) - 1

    @functools.partial(
        pl.kernel,
        out_type=jax.ShapeDtypeStruct((N, D), jnp.float32),
        mesh=_sc_mesh(),
        compiler_params=_sc_params(),
        scratch_types=[
            pltpu.VMEM((cap,), jnp.int32),
            pltpu.VMEM((_KC,), jnp.int32),
            pltpu.VMEM((_KC,), jnp.int32),
            pltpu.VMEM((_KC,), jnp.int32),
            pltpu.VMEM((_KC,), jnp.int32),
            pltpu.VMEM((_KC, D), jnp.float32),
            pltpu.VMEM((_KC, D), jnp.float32),
            pltpu.VMEM((_L,), jnp.int32),
            pltpu.VMEM((_L,), jnp.int32),
            pltpu.VMEM_SHARED((nacc, D), jnp.float32),
            pltpu.SemaphoreType.DMA,
            pltpu.SemaphoreType.DMA,
        ],
    )
    def scat_k(y_hbm, pk_hbm, cnt_hbm, f_hbm, zeros_hbm, out_hbm, pkreg,
               si0, di0, si1, di1, rows0, rows1, cnt_v, f_v, acc_sh,
               sem0, sem1):
        c = lax.axis_index("c")
        s = lax.axis_index("s")
        row0 = pl.multiple_of(s * ach, 8)
        pltpu.sync_copy(zeros_hbm, acc_sh.at[pl.ds(row0, ach)])

        @pl.when(s == _NS - 1)
        def _zt():
            pltpu.sync_copy(zeros_hbm.at[pl.ds(0, atail)],
                            acc_sh.at[pl.ds(half - atail, atail)])

        pltpu.sync_copy(f_hbm, f_v)
        isdeg = f_v[...][0]

        @pl.when(isdeg != 0)
        def _fill_ones():
            one16 = jnp.full((_L,), 1.0, jnp.float32)

            def fill(j, carry):
                for kk in range(D // _L):
                    rows0[j, pl.ds(kk * _L, _L)] = one16
                return carry

            lax.fori_loop(0, _KC, fill, 0)

        plsc.subcore_barrier()

        def unpack(i, si, di):
            for k in range(_KC // _L):
                pk = pkreg[pl.ds(i * _KC + k * _L, _L)]
                si[pl.ds(k * _L, _L)] = pk & mask14
                di[pl.ds(k * _L, _L)] = lax.shift_right_logical(pk, _SH)

        for r in range(2):
            w = 2 * s + r
            pltpu.sync_copy(cnt_hbm.at[pl.ds(w * _L, _L)], cnt_v)
            cvec = cnt_v[...]
            cnt = jnp.where(c == 0, cvec[0], cvec[1])
            nch = lax.div(cnt + (_KC - 1), _KC)
            regbase = pl.multiple_of((c * _NW + w) * cap, 8)
            pltpu.sync_copy(pk_hbm.at[pl.ds(regbase, cap)], pkreg)

            @pl.when(isdeg != 0)
            def _deg_loop():
                def dbody(i, carry):
                    unpack(i, si0, di0)
                    pltpu.sync_copy(rows0, acc_sh.at[di0], add=True)
                    return carry

                lax.fori_loop(0, nch, dbody, 0)

            @pl.when(isdeg == 0)
            def _main_loop():
                @pl.when(nch > 0)
                def _prime():
                    unpack(0, si0, di0)
                    pltpu.async_copy(y_hbm.at[si0], rows0, sem0)

                def pbody(i2, carry):
                    i0 = 2 * i2
                    i1 = i0 + 1

                    @pl.when(i1 < nch)
                    def _g1():
                        unpack(i1, si1, di1)
                        pltpu.async_copy(y_hbm.at[si1], rows1, sem1)

                    @pl.when(i0 < nch)
                    def _s0():
                        pltpu.make_async_copy(y_hbm.at[si0], rows0,
                                              sem0).wait()
                        pltpu.sync_copy(rows0, acc_sh.at[di0], add=True)

                    @pl.when(i0 + 2 < nch)
                    def _g0():
                        unpack(i0 + 2, si0, di0)
                        pltpu.async_copy(y_hbm.at[si0], rows0, sem0)

                    @pl.when(i1 < nch)
                    def _s1():
                        pltpu.make_async_copy(y_hbm.at[si1], rows1,
                                              sem1).wait()
                        pltpu.sync_copy(rows1, acc_sh.at[di1], add=True)

                    return carry

                lax.fori_loop(0, lax.div(nch + 1, 2), pbody, 0)

        plsc.subcore_barrier()
        out0 = pl.multiple_of(c * half + s * ach, 8)
        pltpu.sync_copy(acc_sh.at[pl.ds(row0, ach)],
                        out_hbm.at[pl.ds(out0, ach)])

        @pl.when(s == _NS - 1)
        def _ct():
            pltpu.sync_copy(acc_sh.at[pl.ds(half - atail, atail)],
                            out_hbm.at[pl.ds(c * half + half - atail, atail)])

    return scat_k


_R = 1000


def _dinv_block(degp_ref):
    return lax.rsqrt(degp_ref[...][:, 0:1] + 1.0)


def _tc_linear_body(h_ref, w_ref, degp_ref, fdeg_ref, y_ref):
    dinv = _dinv_block(degp_ref)
    y = jnp.dot(h_ref[...], w_ref[...],
                preferred_element_type=jnp.float32,
                precision=lax.Precision.HIGHEST) * dinv
    y_ref[...] = jnp.where(fdeg_ref[...] > 0.0, 1.0, y)


def _tc_post_body(acc_ref, y_ref, degp_ref, h_ref, b_ref, frelu_ref,
                  fdeg_ref, hn_ref, degn_ref):
    dinv = _dinv_block(degp_ref)
    hp = dinv * (acc_ref[...] + y_ref[...]) + b_ref[...]
    hp = jnp.where(frelu_ref[...] > 0.0, jnp.maximum(hp, 0.0), hp)
    isdeg = fdeg_ref[...] > 0.0
    hn_ref[...] = jnp.where(isdeg, h_ref[...], hp)
    degn_ref[...] = jnp.where(isdeg, acc_ref[...], degp_ref[...])


def _tc_head_body(h_ref, wc_ref, bc_ref, out_ref):
    out_ref[...] = jnp.dot(h_ref[...], wc_ref[...],
                           preferred_element_type=jnp.float32,
                           precision=lax.Precision.HIGHEST) + bc_ref[...]


def _row_spec(D):
    return pl.BlockSpec((_R, D), lambda i: (i, 0))


def _full_spec(shape):
    nd = len(shape)
    return pl.BlockSpec(shape, lambda i, _nd=nd: (0,) * _nd)


def _tc_linear(h, W, degp, fdeg):
    N, D = h.shape
    return pl.pallas_call(
        _tc_linear_body,
        grid=(N // _R,),
        in_specs=[_row_spec(D), _full_spec((D, D)), _row_spec(D),
                  _full_spec((1, D))],
        out_specs=_row_spec(D),
        out_shape=jax.ShapeDtypeStruct((N, D), jnp.float32),
    )(h, W, degp, fdeg)


def _tc_post(acc, y, degp, h, b, frelu, fdeg):
    N, D = y.shape
    return pl.pallas_call(
        _tc_post_body,
        grid=(N // _R,),
        in_specs=[_row_spec(D), _row_spec(D), _row_spec(D), _row_spec(D),
                  _full_spec((1, D)), _full_spec((1, D)), _full_spec((1, D))],
        out_specs=[_row_spec(D), _row_spec(D)],
        out_shape=[jax.ShapeDtypeStruct((N, D), jnp.float32),
                   jax.ShapeDtypeStruct((N, D), jnp.float32)],
    )(acc, y, degp, h, b, frelu, fdeg)


def _tc_head(h, Wc, bc):
    N, D = h.shape
    return pl.pallas_call(
        _tc_head_body,
        grid=(N // _R,),
        in_specs=[_row_spec(D), _full_spec((D, D)), _full_spec((1, D))],
        out_specs=_row_spec(D),
        out_shape=jax.ShapeDtypeStruct((N, D), jnp.float32),
    )(h, Wc, bc)


def kernel(x, edge_index, W1, b1, W2, b2, Wc, bc):
    N, D = x.shape
    E = edge_index.shape[1]
    epw = E // _NW
    src3 = edge_index[0].reshape(_NW, epw // _K, _K)
    dst3 = edge_index[1].reshape(_NW, epw // _K, _K)
    ach, _ = _row_split(N // 2)

    part_fn = _make_part(E, N)
    scat_fn = _make_scatter(E, N, D)

    zacc = jnp.zeros((ach, D), jnp.float32)
    pk, cnts = part_fn(src3, dst3)

    one_row = jnp.ones((1, D), jnp.float32)
    zero_row = jnp.zeros((1, D), jnp.float32)
    w_st = jnp.stack([W1, W1, W2])
    b_st = jnp.stack([zero_row, b1.reshape(1, D), b2.reshape(1, D)])
    frelu_st = jnp.stack([zero_row, one_row, zero_row])
    fdeg_st = jnp.stack([one_row, zero_row, zero_row])
    fsc_st = jnp.stack([jnp.ones((_L,), jnp.int32),
                        jnp.zeros((_L,), jnp.int32),
                        jnp.zeros((_L,), jnp.int32)])

    def step(carry, xs):
        h, degp = carry
        W, b, frelu, fdeg, fsc = xs
        y = _tc_linear(h, W, degp, fdeg)
        acc = scat_fn(y, pk, cnts, fsc, zacc)
        hn, degn = _tc_post(acc, y, degp, h, b, frelu, fdeg)
        return (hn, degn), None

    degp0 = jnp.zeros((N, D), jnp.float32)
    (h, _), _ = lax.scan(step, (x, degp0),
                         (w_st, b_st, frelu_st, fdeg_st, fsc_st))
    out = _tc_head(h, Wc, bc.reshape(1, D))
    return (h, out)
